# baseline (speedup 1.0000x reference)
import functools

import jax
import jax.numpy as jnp
from jax import lax
from jax.experimental import pallas as pl
from jax.experimental.pallas import tpu as pltpu
from jax.experimental.pallas import tpu_sc as plsc

N = 10000
E = 320000
NC, NS = 2, 16
NW = NC * NS
CHUNK = 128
EPW = 80
E_PAD = NW * EPW * CHUNK
N_PAD = 10016
RPT = N_PAD // NS
DUMMY_DST = N

ROWS_BLK = 1000
GRID = N // ROWS_BLK


def _make_sc_aggregate(dp):
    mesh = plsc.VectorSubcoreMesh(core_axis_name="c", subcore_axis_name="s")

    K = 2
    D = 1

    @functools.partial(
        pl.kernel,
        mesh=mesh,
        compiler_params=pltpu.CompilerParams(use_tc_tiling_on_sc=False),
        out_type=jax.ShapeDtypeStruct((NC, N_PAD, dp), jnp.float32),
        scratch_types=[
            pltpu.VMEM((EPW, CHUNK), jnp.int32),
            pltpu.VMEM((EPW, CHUNK), jnp.int32),
            pltpu.VMEM((K, CHUNK, dp), jnp.float32),
            pltpu.VMEM_SHARED((N_PAD, dp), jnp.float32),
            pltpu.SemaphoreType.DMA((K,)),
            pltpu.SemaphoreType.DMA((K,)),
        ],
    )
    def sc_aggregate(proj_hbm, src_hbm, dst_hbm, zeros_hbm, out_hbm,
                     src_v, dst_v, buf, accum, sem_g, sem_s):
        c = lax.axis_index("c")
        s = lax.axis_index("s")
        wid = s * NC + c
        row0 = s * RPT

        pltpu.sync_copy(zeros_hbm.at[pl.ds(row0, RPT)],
                        accum.at[pl.ds(row0, RPT)])
        pltpu.sync_copy(src_hbm.at[wid], src_v)
        pltpu.sync_copy(dst_hbm.at[wid], dst_v)
        plsc.subcore_barrier()

        def start_gather(j, m):
            pltpu.async_copy(proj_hbm.at[src_v.at[j]], buf.at[m], sem_g.at[m])

        def wait_gather(j, m):
            pltpu.make_async_copy(proj_hbm.at[src_v.at[j]], buf.at[m],
                                  sem_g.at[m]).wait()

        def start_scatter(j, m):
            pltpu.async_copy(buf.at[m], accum.at[dst_v.at[j]], sem_s.at[m],
                             add=True)

        def wait_scatter(m):
            pltpu.make_async_copy(zeros_hbm.at[pl.ds(0, CHUNK)], buf.at[m],
                                  sem_s.at[m]).wait()

        for m in range(K):
            start_gather(m, m)

        def head(j, _):
            m = j % K
            wait_gather(j, m)
            start_scatter(j, m)
            return 0

        def steady(j, _):
            m = j % K
            wait_gather(j, m)
            start_scatter(j, m)
            mp = (j - D) % K
            wait_scatter(mp)
            start_gather(j - D + K, mp)
            return 0

        def tail(j, _):
            m = j % K
            wait_gather(j, m)
            start_scatter(j, m)
            wait_scatter((j - D) % K)
            return 0

        lax.fori_loop(0, D, head, 0)
        lax.fori_loop(D, EPW - K + D, steady, 0)
        lax.fori_loop(EPW - K + D, EPW, tail, 0)
        for j in range(EPW - D, EPW):
            wait_scatter(j % K)
        plsc.subcore_barrier()

        pltpu.sync_copy(accum.at[pl.ds(row0, RPT)],
                        out_hbm.at[c].at[pl.ds(row0, RPT)])

    return sc_aggregate


_SC_AGG = {dp: _make_sc_aggregate(dp) for dp in (96, 64, 32, 16)}


def _tc0_body(x_ref, w_ref, ones_ref, o_ref):
    o_ref[...] = (jnp.dot(x_ref[...], w_ref[...],
                          preferred_element_type=jnp.float32) + ones_ref[...])


def _tc0(x, wl0p, ones_row):
    return pl.pallas_call(
        _tc0_body,
        grid=(GRID,),
        in_specs=[
            pl.BlockSpec((ROWS_BLK, 128), lambda r: (r, 0)),
            pl.BlockSpec((128, 96), lambda r: (0, 0)),
            pl.BlockSpec((1, 96), lambda r: (0, 0)),
        ],
        out_specs=pl.BlockSpec((ROWS_BLK, 96), lambda r: (r, 0)),
        out_shape=jax.ShapeDtypeStruct((N, 96), jnp.float32),
    )(x, wl0p, ones_row)


def _tc1_body(acc_ref, x_ref, wr_ref, bl_ref, wl_ref, h_ref, p_ref, cnt_ref):
    s = acc_ref[0] + acc_ref[1]
    cnt = jnp.maximum(s[:, 85:86], 1.0)
    h = (s[:, :85] / cnt
         + jnp.dot(x_ref[...], wr_ref[...], preferred_element_type=jnp.float32)
         + bl_ref[...])
    h_ref[...] = h
    p_ref[...] = jnp.dot(h, wl_ref[...], preferred_element_type=jnp.float32)
    cnt_ref[...] = cnt


def _tc1(acc0, x, wr0t, bl0, wl1p):
    return pl.pallas_call(
        _tc1_body,
        grid=(GRID,),
        in_specs=[
            pl.BlockSpec((NC, ROWS_BLK, 96), lambda r: (0, r, 0)),
            pl.BlockSpec((ROWS_BLK, 128), lambda r: (r, 0)),
            pl.BlockSpec((128, 85), lambda r: (0, 0)),
            pl.BlockSpec((1, 85), lambda r: (0, 0)),
            pl.BlockSpec((85, 64), lambda r: (0, 0)),
        ],
        out_specs=(
            pl.BlockSpec((ROWS_BLK, 85), lambda r: (r, 0)),
            pl.BlockSpec((ROWS_BLK, 64), lambda r: (r, 0)),
            pl.BlockSpec((ROWS_BLK, 1), lambda r: (r, 0)),
        ),
        out_shape=(
            jax.ShapeDtypeStruct((N, 85), jnp.float32),
            jax.ShapeDtypeStruct((N, 64), jnp.float32),
            jax.ShapeDtypeStruct((N, 1), jnp.float32),
        ),
    )(acc0, x, wr0t, bl0, wl1p)


def _make_tc_mid_body(dout_prev):
    def body(acc_ref, cnt_ref, h_ref, wr_ref, bl_ref, wl_ref, ho_ref, p_ref):
        s = acc_ref[0] + acc_ref[1]
        h = (s[:, :dout_prev] / cnt_ref[...]
             + jnp.dot(h_ref[...], wr_ref[...],
                       preferred_element_type=jnp.float32)
             + bl_ref[...])
        ho_ref[...] = h
        p_ref[...] = jnp.dot(h, wl_ref[...], preferred_element_type=jnp.float32)
    return body


def _tc_mid(acc, cnt, h, wrt, bl, wlp, dp_prev, dout_prev, din, dout, dp_next):
    return pl.pallas_call(
        _make_tc_mid_body(dout_prev),
        grid=(GRID,),
        in_specs=[
            pl.BlockSpec((NC, ROWS_BLK, dp_prev), lambda r: (0, r, 0)),
            pl.BlockSpec((ROWS_BLK, 1), lambda r: (r, 0)),
            pl.BlockSpec((ROWS_BLK, din), lambda r: (r, 0)),
            pl.BlockSpec((din, dout), lambda r: (0, 0)),
            pl.BlockSpec((1, dout), lambda r: (0, 0)),
            pl.BlockSpec((dout, dp_next), lambda r: (0, 0)),
        ],
        out_specs=(
            pl.BlockSpec((ROWS_BLK, dout), lambda r: (r, 0)),
            pl.BlockSpec((ROWS_BLK, dp_next), lambda r: (r, 0)),
        ),
        out_shape=(
            jax.ShapeDtypeStruct((N, dout), jnp.float32),
            jax.ShapeDtypeStruct((N, dp_next), jnp.float32),
        ),
    )(acc, cnt, h, wrt, bl, wlp)


def _tc_fin_body(acc_ref, cnt_ref, h_ref, wr_ref, bl_ref, o_ref):
    s = acc_ref[0] + acc_ref[1]
    o_ref[...] = (s / cnt_ref[...]
                  + jnp.dot(h_ref[...], wr_ref[...],
                            preferred_element_type=jnp.float32)
                  + bl_ref[...])


def _tc_fin(acc, cnt, h, wrt, bl):
    return pl.pallas_call(
        _tc_fin_body,
        grid=(GRID,),
        in_specs=[
            pl.BlockSpec((NC, ROWS_BLK, 16), lambda r: (0, r, 0)),
            pl.BlockSpec((ROWS_BLK, 1), lambda r: (r, 0)),
            pl.BlockSpec((ROWS_BLK, 28), lambda r: (r, 0)),
            pl.BlockSpec((28, 16), lambda r: (0, 0)),
            pl.BlockSpec((1, 16), lambda r: (0, 0)),
        ],
        out_specs=pl.BlockSpec((ROWS_BLK, 16), lambda r: (r, 0)),
        out_shape=jax.ShapeDtypeStruct((N, 16), jnp.float32),
    )(acc, cnt, h, wrt, bl)


def kernel(x, edge_index, Wl0, bl0, Wr0, Wl1, bl1, Wr1, Wl2, bl2, Wr2,
           Wl3, bl3, Wr3):
    ei = edge_index.astype(jnp.int32)
    src = jnp.concatenate([ei[0], jnp.zeros((E_PAD - E,), jnp.int32)])
    dst = jnp.concatenate(
        [ei[1], jnp.full((E_PAD - E,), DUMMY_DST, jnp.int32)])
    src3 = src.reshape(NW, EPW, CHUNK)
    dst3 = dst.reshape(NW, EPW, CHUNK)

    wl0p = jnp.pad(Wl0.T, ((0, 0), (0, 96 - 85)))
    ones_row = jnp.zeros((1, 96), jnp.float32).at[0, 85].set(1.0)
    wl1p = jnp.pad(Wl1.T, ((0, 0), (0, 64 - 56)))
    wl2p = jnp.pad(Wl2.T, ((0, 0), (0, 32 - 28)))
    wl3p = jnp.pad(Wl3.T, ((0, 0), (0, 16 - 1)))
    wr3p = jnp.pad(Wr3.T, ((0, 0), (0, 16 - 1)))
    bl3p = jnp.pad(bl3.reshape(1, -1), ((0, 0), (0, 16 - 1)))

    z96 = jnp.zeros((N_PAD, 96), jnp.float32)
    z64 = jnp.zeros((N_PAD, 64), jnp.float32)
    z32 = jnp.zeros((N_PAD, 32), jnp.float32)
    z16 = jnp.zeros((N_PAD, 16), jnp.float32)

    proj0 = _tc0(x, wl0p, ones_row)
    acc0 = _SC_AGG[96](proj0, src3, dst3, z96)
    h1, p1, cnt = _tc1(acc0, x, Wr0.T, bl0.reshape(1, -1), wl1p)
    acc1 = _SC_AGG[64](p1, src3, dst3, z64)
    h2, p2 = _tc_mid(acc1, cnt, h1, Wr1.T, bl1.reshape(1, -1), wl2p,
                     64, 56, 85, 56, 32)
    acc2 = _SC_AGG[32](p2, src3, dst3, z32)
    h3, p3 = _tc_mid(acc2, cnt, h2, Wr2.T, bl2.reshape(1, -1), wl3p,
                     32, 28, 56, 28, 16)
    acc3 = _SC_AGG[16](p3, src3, dst3, z16)
    out = _tc_fin(acc3, cnt, h3, wr3p, bl3p)
    return out[:, :1]


# speedup vs baseline: 1.0656x; 1.0656x over previous
import functools

import jax
import jax.numpy as jnp
from jax import lax
from jax.experimental import pallas as pl
from jax.experimental.pallas import tpu as pltpu
from jax.experimental.pallas import tpu_sc as plsc

N = 10000
E = 320000
NC, NS = 2, 16
NW = NC * NS
CHUNK = 128
EPP = 160
NCHUNKS = NS * EPP
E_PAD = NCHUNKS * CHUNK
N_PAD = 10016
RPT = N_PAD // NS
DUMMY_DST = N

ROWS_BLK = 1000
GRID = N // ROWS_BLK


def _make_sc_aggregate(dp, e0):
    mesh = plsc.VectorSubcoreMesh(core_axis_name="c", subcore_axis_name="s")

    K = 2
    D = 1
    KI = 6
    KA = 3
    e1 = EPP - e0

    @functools.partial(
        pl.kernel,
        mesh=mesh,
        compiler_params=pltpu.CompilerParams(use_tc_tiling_on_sc=False),
        out_type=jax.ShapeDtypeStruct((NC, N_PAD, dp), jnp.float32),
        scratch_types=[
            pltpu.VMEM((KI, 2, CHUNK), jnp.int32),
            pltpu.VMEM((K, CHUNK, dp), jnp.float32),
            pltpu.VMEM_SHARED((N_PAD, dp), jnp.float32),
            pltpu.SemaphoreType.DMA((KI,)),
            pltpu.SemaphoreType.DMA((K,)),
            pltpu.SemaphoreType.DMA((K,)),
        ],
    )
    def sc_aggregate(proj_hbm, edges_hbm, zeros_hbm, out_hbm,
                     idx_v, buf, accum, sem_i, sem_g, sem_s):
        c = lax.axis_index("c")
        s = lax.axis_index("s")
        row0 = s * RPT
        base = jnp.where(c == 0, s * e0, 16 * e0 + s * e1)
        count = jnp.where(c == 0, e0, e1)

        pltpu.sync_copy(zeros_hbm.at[pl.ds(row0, RPT)],
                        accum.at[pl.ds(row0, RPT)])
        plsc.subcore_barrier()

        def start_idx(h):
            mi = h % KI
            pltpu.async_copy(edges_hbm.at[base + h], idx_v.at[mi],
                             sem_i.at[mi])

        def wait_idx(h):
            mi = h % KI
            pltpu.make_async_copy(edges_hbm.at[base + h], idx_v.at[mi],
                                  sem_i.at[mi]).wait()

        def start_gather(j, m):
            pltpu.async_copy(proj_hbm.at[idx_v.at[j % KI, 0]], buf.at[m],
                             sem_g.at[m])

        def wait_gather(j, m):
            pltpu.make_async_copy(proj_hbm.at[idx_v.at[j % KI, 0]],
                                  buf.at[m], sem_g.at[m]).wait()

        def start_scatter(j, m):
            pltpu.async_copy(buf.at[m], accum.at[idx_v.at[j % KI, 1]],
                             sem_s.at[m], add=True)

        def wait_scatter(m):
            pltpu.make_async_copy(zeros_hbm.at[pl.ds(0, CHUNK)], buf.at[m],
                                  sem_s.at[m]).wait()

        for h in range(KA):
            start_idx(h)
        for g in range(K):
            wait_idx(g)
            start_gather(g, g)

        def body(j, _):
            m = j % K
            wait_gather(j, m)
            h = j + KA
            lax.cond(h < count, lambda: start_idx(h), lambda: None)
            start_scatter(j, m)

            def refill():
                mp = (j - D) % K
                wait_scatter(mp)
                g = j - D + K
                lax.cond(g < count,
                         lambda: (wait_idx(g), start_gather(g, mp))[1],
                         lambda: None)

            lax.cond(j >= D, refill, lambda: None)
            return 0

        lax.fori_loop(0, count, body, 0)
        for d in range(D):
            wait_scatter((count - 1 - d) % K)
        plsc.subcore_barrier()

        pltpu.sync_copy(accum.at[pl.ds(row0, RPT)],
                        out_hbm.at[c].at[pl.ds(row0, RPT)])

    return sc_aggregate


_E0 = {96: 38, 64: 42, 32: 53, 16: 64}
_SC_AGG = {dp: _make_sc_aggregate(dp, _E0[dp]) for dp in (96, 64, 32, 16)}


def _tc0_body(x_ref, w_ref, ones_ref, o_ref):
    o_ref[...] = (jnp.dot(x_ref[...], w_ref[...],
                          preferred_element_type=jnp.float32) + ones_ref[...])


def _tc0(x, wl0p, ones_row):
    return pl.pallas_call(
        _tc0_body,
        grid=(GRID,),
        in_specs=[
            pl.BlockSpec((ROWS_BLK, 128), lambda r: (r, 0)),
            pl.BlockSpec((128, 96), lambda r: (0, 0)),
            pl.BlockSpec((1, 96), lambda r: (0, 0)),
        ],
        out_specs=pl.BlockSpec((ROWS_BLK, 96), lambda r: (r, 0)),
        out_shape=jax.ShapeDtypeStruct((N, 96), jnp.float32),
    )(x, wl0p, ones_row)


def _tc1_body(acc_ref, x_ref, wr_ref, bl_ref, wl_ref, h_ref, p_ref, cnt_ref):
    s = acc_ref[0] + acc_ref[1]
    cnt = jnp.maximum(s[:, 85:86], 1.0)
    h = (s[:, :85] / cnt
         + jnp.dot(x_ref[...], wr_ref[...], preferred_element_type=jnp.float32)
         + bl_ref[...])
    h_ref[...] = h
    p_ref[...] = jnp.dot(h, wl_ref[...], preferred_element_type=jnp.float32)
    cnt_ref[...] = cnt


def _tc1(acc0, x, wr0t, bl0, wl1p):
    return pl.pallas_call(
        _tc1_body,
        grid=(GRID,),
        in_specs=[
            pl.BlockSpec((NC, ROWS_BLK, 96), lambda r: (0, r, 0)),
            pl.BlockSpec((ROWS_BLK, 128), lambda r: (r, 0)),
            pl.BlockSpec((128, 85), lambda r: (0, 0)),
            pl.BlockSpec((1, 85), lambda r: (0, 0)),
            pl.BlockSpec((85, 64), lambda r: (0, 0)),
        ],
        out_specs=(
            pl.BlockSpec((ROWS_BLK, 85), lambda r: (r, 0)),
            pl.BlockSpec((ROWS_BLK, 64), lambda r: (r, 0)),
            pl.BlockSpec((ROWS_BLK, 1), lambda r: (r, 0)),
        ),
        out_shape=(
            jax.ShapeDtypeStruct((N, 85), jnp.float32),
            jax.ShapeDtypeStruct((N, 64), jnp.float32),
            jax.ShapeDtypeStruct((N, 1), jnp.float32),
        ),
    )(acc0, x, wr0t, bl0, wl1p)


def _make_tc_mid_body(dout_prev):
    def body(acc_ref, cnt_ref, h_ref, wr_ref, bl_ref, wl_ref, ho_ref, p_ref):
        s = acc_ref[0] + acc_ref[1]
        h = (s[:, :dout_prev] / cnt_ref[...]
             + jnp.dot(h_ref[...], wr_ref[...],
                       preferred_element_type=jnp.float32)
             + bl_ref[...])
        ho_ref[...] = h
        p_ref[...] = jnp.dot(h, wl_ref[...], preferred_element_type=jnp.float32)
    return body


def _tc_mid(acc, cnt, h, wrt, bl, wlp, dp_prev, dout_prev, din, dout, dp_next):
    return pl.pallas_call(
        _make_tc_mid_body(dout_prev),
        grid=(GRID,),
        in_specs=[
            pl.BlockSpec((NC, ROWS_BLK, dp_prev), lambda r: (0, r, 0)),
            pl.BlockSpec((ROWS_BLK, 1), lambda r: (r, 0)),
            pl.BlockSpec((ROWS_BLK, din), lambda r: (r, 0)),
            pl.BlockSpec((din, dout), lambda r: (0, 0)),
            pl.BlockSpec((1, dout), lambda r: (0, 0)),
            pl.BlockSpec((dout, dp_next), lambda r: (0, 0)),
        ],
        out_specs=(
            pl.BlockSpec((ROWS_BLK, dout), lambda r: (r, 0)),
            pl.BlockSpec((ROWS_BLK, dp_next), lambda r: (r, 0)),
        ),
        out_shape=(
            jax.ShapeDtypeStruct((N, dout), jnp.float32),
            jax.ShapeDtypeStruct((N, dp_next), jnp.float32),
        ),
    )(acc, cnt, h, wrt, bl, wlp)


def _tc_fin_body(acc_ref, cnt_ref, h_ref, wr_ref, bl_ref, o_ref):
    s = acc_ref[0] + acc_ref[1]
    o_ref[...] = (s / cnt_ref[...]
                  + jnp.dot(h_ref[...], wr_ref[...],
                            preferred_element_type=jnp.float32)
                  + bl_ref[...])


def _tc_fin(acc, cnt, h, wrt, bl):
    return pl.pallas_call(
        _tc_fin_body,
        grid=(GRID,),
        in_specs=[
            pl.BlockSpec((NC, ROWS_BLK, 16), lambda r: (0, r, 0)),
            pl.BlockSpec((ROWS_BLK, 1), lambda r: (r, 0)),
            pl.BlockSpec((ROWS_BLK, 28), lambda r: (r, 0)),
            pl.BlockSpec((28, 16), lambda r: (0, 0)),
            pl.BlockSpec((1, 16), lambda r: (0, 0)),
        ],
        out_specs=pl.BlockSpec((ROWS_BLK, 16), lambda r: (r, 0)),
        out_shape=jax.ShapeDtypeStruct((N, 16), jnp.float32),
    )(acc, cnt, h, wrt, bl)


def kernel(x, edge_index, Wl0, bl0, Wr0, Wl1, bl1, Wr1, Wl2, bl2, Wr2,
           Wl3, bl3, Wr3):
    ei = edge_index.astype(jnp.int32)
    src = jnp.concatenate([ei[0], jnp.zeros((E_PAD - E,), jnp.int32)])
    dst = jnp.concatenate(
        [ei[1], jnp.full((E_PAD - E,), DUMMY_DST, jnp.int32)])
    edges3 = jnp.stack(
        [src.reshape(NCHUNKS, CHUNK), dst.reshape(NCHUNKS, CHUNK)], axis=1)

    wl0p = jnp.pad(Wl0.T, ((0, 0), (0, 96 - 85)))
    ones_row = jnp.zeros((1, 96), jnp.float32).at[0, 85].set(1.0)
    wl1p = jnp.pad(Wl1.T, ((0, 0), (0, 64 - 56)))
    wl2p = jnp.pad(Wl2.T, ((0, 0), (0, 32 - 28)))
    wl3p = jnp.pad(Wl3.T, ((0, 0), (0, 16 - 1)))
    wr3p = jnp.pad(Wr3.T, ((0, 0), (0, 16 - 1)))
    bl3p = jnp.pad(bl3.reshape(1, -1), ((0, 0), (0, 16 - 1)))

    z96 = jnp.zeros((N_PAD, 96), jnp.float32)
    z64 = jnp.zeros((N_PAD, 64), jnp.float32)
    z32 = jnp.zeros((N_PAD, 32), jnp.float32)
    z16 = jnp.zeros((N_PAD, 16), jnp.float32)

    proj0 = _tc0(x, wl0p, ones_row)
    acc0 = _SC_AGG[96](proj0, edges3, z96)
    h1, p1, cnt = _tc1(acc0, x, Wr0.T, bl0.reshape(1, -1), wl1p)
    acc1 = _SC_AGG[64](p1, edges3, z64)
    h2, p2 = _tc_mid(acc1, cnt, h1, Wr1.T, bl1.reshape(1, -1), wl2p,
                     64, 56, 85, 56, 32)
    acc2 = _SC_AGG[32](p2, edges3, z32)
    h3, p3 = _tc_mid(acc2, cnt, h2, Wr2.T, bl2.reshape(1, -1), wl3p,
                     32, 28, 56, 28, 16)
    acc3 = _SC_AGG[16](p3, edges3, z16)
    out = _tc_fin(acc3, cnt, h3, wr3p, bl3p)
    return out[:, :1]


# speedup vs baseline: 1.2961x; 1.2163x over previous
import functools

import jax
import jax.numpy as jnp
from jax import lax
from jax.experimental import pallas as pl
from jax.experimental.pallas import tpu as pltpu
from jax.experimental.pallas import tpu_sc as plsc

N = 10000
E = 320000
NC, NS = 2, 16
NW = NC * NS
CHUNK = 128
EPP = 160
NCHUNKS = NS * EPP
E_PAD = NCHUNKS * CHUNK
N_PAD = 10016
RPT = N_PAD // NS
DUMMY_DST = N

ROWS_BLK = 1000
GRID = N // ROWS_BLK


def _make_sc_aggregate(dp, e0):
    mesh = plsc.VectorSubcoreMesh(core_axis_name="c", subcore_axis_name="s")

    K = 2
    D = 1
    KI = 6
    KA = 3
    e1 = EPP - e0

    @functools.partial(
        pl.kernel,
        mesh=mesh,
        compiler_params=pltpu.CompilerParams(use_tc_tiling_on_sc=False),
        out_type=jax.ShapeDtypeStruct((NC, N_PAD, dp), jnp.float32),
        scratch_types=[
            pltpu.VMEM((KI, 2, CHUNK), jnp.int32),
            pltpu.VMEM((K, CHUNK, dp), jnp.float32),
            pltpu.VMEM_SHARED((N_PAD, dp), jnp.float32),
            pltpu.SemaphoreType.DMA((KI,)),
            pltpu.SemaphoreType.DMA((K,)),
            pltpu.SemaphoreType.DMA((K,)),
        ],
    )
    def sc_aggregate(proj_hbm, edges_hbm, zeros_hbm, out_hbm,
                     idx_v, buf, accum, sem_i, sem_g, sem_s):
        c = lax.axis_index("c")
        s = lax.axis_index("s")
        row0 = s * RPT
        base = jnp.where(c == 0, s * e0, 16 * e0 + s * e1)
        count = jnp.where(c == 0, e0, e1)

        pltpu.sync_copy(zeros_hbm.at[pl.ds(row0, RPT)],
                        accum.at[pl.ds(row0, RPT)])
        plsc.subcore_barrier()

        def start_idx(h):
            mi = h % KI
            pltpu.async_copy(edges_hbm.at[base + h], idx_v.at[mi],
                             sem_i.at[mi])

        def wait_idx(h):
            mi = h % KI
            pltpu.make_async_copy(edges_hbm.at[base + h], idx_v.at[mi],
                                  sem_i.at[mi]).wait()

        def start_gather(j, m):
            pltpu.async_copy(proj_hbm.at[idx_v.at[j % KI, 0]], buf.at[m],
                             sem_g.at[m])

        def wait_gather(j, m):
            pltpu.make_async_copy(proj_hbm.at[idx_v.at[j % KI, 0]],
                                  buf.at[m], sem_g.at[m]).wait()

        def start_scatter(j, m):
            pltpu.async_copy(buf.at[m], accum.at[idx_v.at[j % KI, 1]],
                             sem_s.at[m], add=True)

        def wait_scatter(m):
            pltpu.make_async_copy(zeros_hbm.at[pl.ds(0, CHUNK)], buf.at[m],
                                  sem_s.at[m]).wait()

        for h in range(KA):
            start_idx(h)
        for g in range(K):
            wait_idx(g)
            start_gather(g, g)

        def body(j, _):
            m = j % K
            wait_gather(j, m)
            h = j + KA
            lax.cond(h < count, lambda: start_idx(h), lambda: None)
            start_scatter(j, m)

            def refill():
                mp = (j - D) % K
                wait_scatter(mp)
                g = j - D + K
                lax.cond(g < count,
                         lambda: (wait_idx(g), start_gather(g, mp))[1],
                         lambda: None)

            lax.cond(j >= D, refill, lambda: None)
            return 0

        lax.fori_loop(0, count, body, 0)
        for d in range(D):
            wait_scatter((count - 1 - d) % K)
        plsc.subcore_barrier()

        pltpu.sync_copy(accum.at[pl.ds(row0, RPT)],
                        out_hbm.at[c].at[pl.ds(row0, RPT)])

    return sc_aggregate


_E0 = {96: 122, 64: 118, 32: 107, 16: 96}
_SC_AGG = {dp: _make_sc_aggregate(dp, _E0[dp]) for dp in (96, 64, 32, 16)}


def _tc0_body(x_ref, w_ref, ones_ref, o_ref):
    o_ref[...] = (jnp.dot(x_ref[...], w_ref[...],
                          preferred_element_type=jnp.float32) + ones_ref[...])


def _tc0(x, wl0p, ones_row):
    return pl.pallas_call(
        _tc0_body,
        grid=(GRID,),
        in_specs=[
            pl.BlockSpec((ROWS_BLK, 128), lambda r: (r, 0)),
            pl.BlockSpec((128, 96), lambda r: (0, 0)),
            pl.BlockSpec((1, 96), lambda r: (0, 0)),
        ],
        out_specs=pl.BlockSpec((ROWS_BLK, 96), lambda r: (r, 0)),
        out_shape=jax.ShapeDtypeStruct((N, 96), jnp.float32),
    )(x, wl0p, ones_row)


def _tc1_body(acc_ref, x_ref, wr_ref, bl_ref, wl_ref, h_ref, p_ref, cnt_ref):
    s = acc_ref[0] + acc_ref[1]
    cnt = jnp.maximum(s[:, 85:86], 1.0)
    h = (s[:, :85] / cnt
         + jnp.dot(x_ref[...], wr_ref[...], preferred_element_type=jnp.float32)
         + bl_ref[...])
    h_ref[...] = h
    p_ref[...] = jnp.dot(h, wl_ref[...], preferred_element_type=jnp.float32)
    cnt_ref[...] = cnt


def _tc1(acc0, x, wr0t, bl0, wl1p):
    return pl.pallas_call(
        _tc1_body,
        grid=(GRID,),
        in_specs=[
            pl.BlockSpec((NC, ROWS_BLK, 96), lambda r: (0, r, 0)),
            pl.BlockSpec((ROWS_BLK, 128), lambda r: (r, 0)),
            pl.BlockSpec((128, 85), lambda r: (0, 0)),
            pl.BlockSpec((1, 85), lambda r: (0, 0)),
            pl.BlockSpec((85, 64), lambda r: (0, 0)),
        ],
        out_specs=(
            pl.BlockSpec((ROWS_BLK, 85), lambda r: (r, 0)),
            pl.BlockSpec((ROWS_BLK, 64), lambda r: (r, 0)),
            pl.BlockSpec((ROWS_BLK, 1), lambda r: (r, 0)),
        ),
        out_shape=(
            jax.ShapeDtypeStruct((N, 85), jnp.float32),
            jax.ShapeDtypeStruct((N, 64), jnp.float32),
            jax.ShapeDtypeStruct((N, 1), jnp.float32),
        ),
    )(acc0, x, wr0t, bl0, wl1p)


def _make_tc_mid_body(dout_prev):
    def body(acc_ref, cnt_ref, h_ref, wr_ref, bl_ref, wl_ref, ho_ref, p_ref):
        s = acc_ref[0] + acc_ref[1]
        h = (s[:, :dout_prev] / cnt_ref[...]
             + jnp.dot(h_ref[...], wr_ref[...],
                       preferred_element_type=jnp.float32)
             + bl_ref[...])
        ho_ref[...] = h
        p_ref[...] = jnp.dot(h, wl_ref[...], preferred_element_type=jnp.float32)
    return body


def _tc_mid(acc, cnt, h, wrt, bl, wlp, dp_prev, dout_prev, din, dout, dp_next):
    return pl.pallas_call(
        _make_tc_mid_body(dout_prev),
        grid=(GRID,),
        in_specs=[
            pl.BlockSpec((NC, ROWS_BLK, dp_prev), lambda r: (0, r, 0)),
            pl.BlockSpec((ROWS_BLK, 1), lambda r: (r, 0)),
            pl.BlockSpec((ROWS_BLK, din), lambda r: (r, 0)),
            pl.BlockSpec((din, dout), lambda r: (0, 0)),
            pl.BlockSpec((1, dout), lambda r: (0, 0)),
            pl.BlockSpec((dout, dp_next), lambda r: (0, 0)),
        ],
        out_specs=(
            pl.BlockSpec((ROWS_BLK, dout), lambda r: (r, 0)),
            pl.BlockSpec((ROWS_BLK, dp_next), lambda r: (r, 0)),
        ),
        out_shape=(
            jax.ShapeDtypeStruct((N, dout), jnp.float32),
            jax.ShapeDtypeStruct((N, dp_next), jnp.float32),
        ),
    )(acc, cnt, h, wrt, bl, wlp)


def _tc_fin_body(acc_ref, cnt_ref, h_ref, wr_ref, bl_ref, o_ref):
    s = acc_ref[0] + acc_ref[1]
    o_ref[...] = (s / cnt_ref[...]
                  + jnp.dot(h_ref[...], wr_ref[...],
                            preferred_element_type=jnp.float32)
                  + bl_ref[...])


def _tc_fin(acc, cnt, h, wrt, bl):
    return pl.pallas_call(
        _tc_fin_body,
        grid=(GRID,),
        in_specs=[
            pl.BlockSpec((NC, ROWS_BLK, 16), lambda r: (0, r, 0)),
            pl.BlockSpec((ROWS_BLK, 1), lambda r: (r, 0)),
            pl.BlockSpec((ROWS_BLK, 28), lambda r: (r, 0)),
            pl.BlockSpec((28, 16), lambda r: (0, 0)),
            pl.BlockSpec((1, 16), lambda r: (0, 0)),
        ],
        out_specs=pl.BlockSpec((ROWS_BLK, 16), lambda r: (r, 0)),
        out_shape=jax.ShapeDtypeStruct((N, 16), jnp.float32),
    )(acc, cnt, h, wrt, bl)


def kernel(x, edge_index, Wl0, bl0, Wr0, Wl1, bl1, Wr1, Wl2, bl2, Wr2,
           Wl3, bl3, Wr3):
    ei = edge_index.astype(jnp.int32)
    src = jnp.concatenate([ei[0], jnp.zeros((E_PAD - E,), jnp.int32)])
    dst = jnp.concatenate(
        [ei[1], jnp.full((E_PAD - E,), DUMMY_DST, jnp.int32)])
    edges3 = jnp.stack(
        [src.reshape(NCHUNKS, CHUNK), dst.reshape(NCHUNKS, CHUNK)], axis=1)

    wl0p = jnp.pad(Wl0.T, ((0, 0), (0, 96 - 85)))
    ones_row = jnp.zeros((1, 96), jnp.float32).at[0, 85].set(1.0)
    wl1p = jnp.pad(Wl1.T, ((0, 0), (0, 64 - 56)))
    wl2p = jnp.pad(Wl2.T, ((0, 0), (0, 32 - 28)))
    wl3p = jnp.pad(Wl3.T, ((0, 0), (0, 16 - 1)))
    wr3p = jnp.pad(Wr3.T, ((0, 0), (0, 16 - 1)))
    bl3p = jnp.pad(bl3.reshape(1, -1), ((0, 0), (0, 16 - 1)))

    z96 = jnp.zeros((N_PAD, 96), jnp.float32)
    z64 = jnp.zeros((N_PAD, 64), jnp.float32)
    z32 = jnp.zeros((N_PAD, 32), jnp.float32)
    z16 = jnp.zeros((N_PAD, 16), jnp.float32)

    proj0 = _tc0(x, wl0p, ones_row)
    acc0 = _SC_AGG[96](proj0, edges3, z96)
    h1, p1, cnt = _tc1(acc0, x, Wr0.T, bl0.reshape(1, -1), wl1p)
    acc1 = _SC_AGG[64](p1, edges3, z64)
    h2, p2 = _tc_mid(acc1, cnt, h1, Wr1.T, bl1.reshape(1, -1), wl2p,
                     64, 56, 85, 56, 32)
    acc2 = _SC_AGG[32](p2, edges3, z32)
    h3, p3 = _tc_mid(acc2, cnt, h2, Wr2.T, bl2.reshape(1, -1), wl3p,
                     32, 28, 56, 28, 16)
    acc3 = _SC_AGG[16](p3, edges3, z16)
    out = _tc_fin(acc3, cnt, h3, wr3p, bl3p)
    return out[:, :1]
